# Initial kernel scaffold; baseline (speedup 1.0000x reference)
import jax
import jax.numpy as jnp
from jax.experimental import pallas as pl


def kernel(atom_features, atom_edge_index, edge_attr, edge_sh, atom_res_batch, res_features, res_edge_index, res_edge_features, W1, b1, W2, Wself, Wr1, Wr2, Wr3, Wz1, Wz2, Wm):
    raise NotImplementedError("write your pallas kernel here")


# speedup vs baseline: 2.7015x; 2.7015x over previous
import functools

import jax
import jax.numpy as jnp
from jax import lax
from jax.experimental import pallas as pl
from jax.experimental.pallas import tpu as pltpu
from jax.experimental.pallas import tpu_sc as plsc

NA = 10000
EA = 320000
NR = 1000
ER = 30000
DA = 56
HE = 16
SH = 4
CS = 128
CZ = 128

NC = 2
NS = 16
NW = NC * NS

NAP = 10240
ERP = 30720
NRP = 1024

_K = 80

_f32 = jnp.float32


def _dot(a, b):
    return jnp.dot(a, b, preferred_element_type=_f32)


def _ln_in(x):
    m = jnp.mean(x, axis=-1, keepdims=True)
    v = jnp.var(x, axis=-1, keepdims=True)
    return (x - m) / jnp.sqrt(v + 1e-5)


def _tck_pre_body(x_ref, w1_ref, wself_ref, ab_ref, xs_ref):
    x = x_ref[:NA, :]
    wab = jnp.concatenate([w1_ref[:DA, :], w1_ref[DA:2 * DA, :]], axis=1)
    ab_ref[...] = _dot(x, wab)
    xs_ref[...] = _dot(x, wself_ref[...])


def _tck_pre(x_pad, w1, wself):
    return pl.pallas_call(
        _tck_pre_body,
        out_shape=[
            jax.ShapeDtypeStruct((NA, 2 * 64), _f32),
            jax.ShapeDtypeStruct((NA, DA), _f32),
        ],
    )(x_pad, w1, wself)


_CE_BLK = 8000


def _tck_c_body(ea_ref, sh_ref, wc_ref, wd_ref, b1_ref, c_ref):
    c_ref[...] = (_dot(ea_ref[...], wc_ref[...]) + _dot(sh_ref[...], wd_ref[...])
                  + b1_ref[...])


def _tck_c(edge_attr, edge_sh, wc, wd, b1):
    nblk = EA // _CE_BLK
    return pl.pallas_call(
        _tck_c_body,
        grid=(nblk,),
        in_specs=[
            pl.BlockSpec((_CE_BLK, HE), lambda i: (i, 0)),
            pl.BlockSpec((_CE_BLK, SH), lambda i: (i, 0)),
            pl.BlockSpec((HE, 64), lambda i: (0, 0)),
            pl.BlockSpec((SH, 64), lambda i: (0, 0)),
            pl.BlockSpec((1, 64), lambda i: (0, 0)),
        ],
        out_specs=pl.BlockSpec((_CE_BLK, 64), lambda i: (i, 0)),
        out_shape=jax.ShapeDtypeStruct((EA, 64), _f32),
    )(edge_attr, edge_sh, wc, wd, b1)


def _tck_x_body(h0_ref, h1_ref, xs_ref, w2_ref, out_ref):
    agg = _dot(h0_ref[:NA, :] + h1_ref[:NA, :], w2_ref[...])
    out_ref[:NA, :] = xs_ref[...] + agg
    out_ref[NA:, :] = jnp.zeros((NAP - NA, DA), _f32)


def _tck_x(h0, h1, xs, w2):
    return pl.pallas_call(
        _tck_x_body,
        out_shape=jax.ShapeDtypeStruct((NAP, DA), _f32),
    )(h0, h1, xs, w2)


def _tck_res1_body(s_ref, r0_ref, r1_ref, wr1_ref, wr2_ref, wr3_ref,
                   wza_ref, wzb_ref, s1_ref, as_ref, bs_ref):
    ru = r0_ref[:NR, :] + r1_ref[:NR, :]
    ru = jnp.maximum(_dot(ru, wr1_ref[...]), 0.0)
    ru = _dot(jnp.maximum(_dot(ru, wr2_ref[...]), 0.0), wr3_ref[...])
    s1 = _ln_in(s_ref[...] + ru)
    s1_ref[...] = s1
    as_ref[...] = _dot(s1, wza_ref[...])
    bs_ref[...] = _dot(s1, wzb_ref[...])


def _tck_res1(s, r0, r1, wr1, wr2, wr3, wza, wzb):
    return pl.pallas_call(
        _tck_res1_body,
        out_shape=[
            jax.ShapeDtypeStruct((NR, CS), _f32),
            jax.ShapeDtypeStruct((NR, CS), _f32),
            jax.ShapeDtypeStruct((NR, CS), _f32),
        ],
    )(s, r0, r1, wr1, wr2, wr3, wza, wzb)


_ZE_BLK = 3840


def _tck_cz_body(z_ref, wzc_ref, cz_ref):
    cz = _dot(z_ref[...], wzc_ref[...])
    base = pl.program_id(0) * _ZE_BLK
    rows = base + lax.broadcasted_iota(jnp.int32, (_ZE_BLK, 1), 0)
    cz_ref[...] = jnp.where(rows < ER, cz, -1e9)


def _tck_cz(z_pad, wzc):
    nblk = ERP // _ZE_BLK
    return pl.pallas_call(
        _tck_cz_body,
        grid=(nblk,),
        in_specs=[
            pl.BlockSpec((_ZE_BLK, CZ), lambda i: (i, 0)),
            pl.BlockSpec((CZ, CZ), lambda i: (0, 0)),
        ],
        out_specs=pl.BlockSpec((_ZE_BLK, CZ), lambda i: (i, 0)),
        out_shape=jax.ShapeDtypeStruct((ERP, CZ), _f32),
    )(z_pad, wzc)


def _tck_em_body(g_ref, z_ref, wz2_ref, wm_ref, zn_ref, t_ref):
    em = _dot(g_ref[...], wz2_ref[...])
    zn_ref[...] = z_ref[...] + em
    t_ref[...] = jnp.maximum(_dot(em, wm_ref[...]), 0.0)


def _tck_em(g, z_pad, wz2, wm):
    nblk = ERP // _ZE_BLK
    return pl.pallas_call(
        _tck_em_body,
        grid=(nblk,),
        in_specs=[
            pl.BlockSpec((_ZE_BLK, CZ), lambda i: (i, 0)),
            pl.BlockSpec((_ZE_BLK, CZ), lambda i: (i, 0)),
            pl.BlockSpec((CZ, CZ), lambda i: (0, 0)),
            pl.BlockSpec((CZ, CS), lambda i: (0, 0)),
        ],
        out_specs=[
            pl.BlockSpec((_ZE_BLK, CZ), lambda i: (i, 0)),
            pl.BlockSpec((_ZE_BLK, CS), lambda i: (i, 0)),
        ],
        out_shape=[
            jax.ShapeDtypeStruct((ERP, CZ), _f32),
            jax.ShapeDtypeStruct((ERP, CS), _f32),
        ],
    )(g, z_pad, wz2, wm)


def _tck_s_body(s1_ref, p0_ref, p1_ref, out_ref):
    out_ref[...] = _ln_in(s1_ref[...] + p0_ref[:NR, :] + p1_ref[:NR, :])


def _tck_s(s1, p0, p1):
    return pl.pallas_call(
        _tck_s_body,
        out_shape=jax.ShapeDtypeStruct((NR, CS), _f32),
    )(s1, p0, p1)


@functools.cache
def _sc_mesh():
    return plsc.VectorSubcoreMesh(core_axis_name="c", subcore_axis_name="s",
                                  num_cores=NC, num_subcores=NS)


def _relu_sum3(ra, rb, rc, ngrp):
    def row(j, carry):
        for t in range(ngrp):
            sl = pl.ds(t * 16, 16)
            v = ra[j, sl] + rb[j, sl] + rc[j, sl]
            ra[j, sl] = jnp.maximum(v, 0.0)
        return carry
    lax.fori_loop(0, _K, row, 0)


def _sc_atom(ab_tab, c_edges, src, dst, zinit):
    epw = EA // NW
    nch = epw // _K

    @functools.partial(
        pl.kernel,
        out_type=jax.ShapeDtypeStruct((NC, NAP, 64), _f32),
        mesh=_sc_mesh(),
        scratch_types=[
            pltpu.VMEM((_K,), jnp.int32),
            pltpu.VMEM((_K,), jnp.int32),
            pltpu.VMEM((_K, 128), _f32),
            pltpu.VMEM((_K, 128), _f32),
            pltpu.VMEM((_K, 64), _f32),
            pltpu.SemaphoreType.DMA,
            pltpu.VMEM_SHARED((NAP, 64), _f32),
        ],
    )
    def k(ab_hbm, c_hbm, src_hbm, dst_hbm, z_hbm, out_hbm,
          si, di, ra, rb, rc, sem, acc):
        cid = lax.axis_index("c")
        sid = lax.axis_index("s")

        @pl.when(sid == 0)
        def _():
            pltpu.sync_copy(z_hbm, acc)
        plsc.subcore_barrier()

        ebase = (cid * NS + sid) * epw

        def chunk(i, carry):
            off = ebase + i * _K
            pltpu.sync_copy(src_hbm.at[pl.ds(off, _K)], si)
            pltpu.sync_copy(dst_hbm.at[pl.ds(off, _K)], di)
            cpa = pltpu.async_copy(ab_hbm.at[si], ra, sem)
            cpb = pltpu.async_copy(ab_hbm.at[di], rb, sem)
            pltpu.sync_copy(c_hbm.at[pl.ds(off, _K)], rc)
            cpa.wait()
            cpb.wait()

            def row(j, c2):
                for t in range(4):
                    sl = pl.ds(t * 16, 16)
                    slb = pl.ds(64 + t * 16, 16)
                    v = ra[j, sl] + rb[j, slb] + rc[j, sl]
                    rc[j, sl] = jnp.maximum(v, 0.0)
                return c2

            lax.fori_loop(0, _K, row, 0)
            pltpu.sync_copy(rc, acc.at[di], add=True)
            return carry

        lax.fori_loop(0, nch, chunk, 0)
        plsc.subcore_barrier()
        rps = NAP // NS
        pltpu.sync_copy(acc.at[pl.ds(sid * rps, rps)],
                        out_hbm.at[cid, pl.ds(sid * rps, rps)])

    return k(ab_tab, c_edges, src, dst, zinit)


def _sc_scatter(rows, idx, zinit, nin, d):
    rpw = nin // NW
    nch = rpw // _K

    @functools.partial(
        pl.kernel,
        out_type=jax.ShapeDtypeStruct((NC, NRP, d), _f32),
        mesh=_sc_mesh(),
        scratch_types=[
            pltpu.VMEM((_K,), jnp.int32),
            pltpu.VMEM((_K, d), _f32),
            pltpu.VMEM_SHARED((NRP, d), _f32),
        ],
    )
    def k(rows_hbm, idx_hbm, z_hbm, out_hbm, di, rr, acc):
        cid = lax.axis_index("c")
        sid = lax.axis_index("s")

        @pl.when(sid == 0)
        def _():
            pltpu.sync_copy(z_hbm, acc)
        plsc.subcore_barrier()

        rbase = (cid * NS + sid) * rpw

        def chunk(i, carry):
            off = rbase + i * _K
            pltpu.sync_copy(idx_hbm.at[pl.ds(off, _K)], di)
            pltpu.sync_copy(rows_hbm.at[pl.ds(off, _K)], rr)
            pltpu.sync_copy(rr, acc.at[di], add=True)
            return carry

        lax.fori_loop(0, nch, chunk, 0)
        plsc.subcore_barrier()

        rps = NRP // NS
        pltpu.sync_copy(acc.at[pl.ds(sid * rps, rps)],
                        out_hbm.at[cid, pl.ds(sid * rps, rps)])

    return k(rows, idx, zinit)


def _sc_resedge(as_tab, bs_tab, cz_edges, rs, rd):
    epw = ERP // NW
    nch = epw // _K

    @functools.partial(
        pl.kernel,
        out_type=jax.ShapeDtypeStruct((ERP, CZ), _f32),
        mesh=_sc_mesh(),
        scratch_types=[
            pltpu.VMEM((_K,), jnp.int32),
            pltpu.VMEM((_K,), jnp.int32),
            pltpu.VMEM((_K, CZ), _f32),
            pltpu.VMEM((_K, CZ), _f32),
            pltpu.VMEM((_K, CZ), _f32),
            pltpu.SemaphoreType.DMA,
        ],
    )
    def k(a_hbm, b_hbm, c_hbm, rs_hbm, rd_hbm, g_hbm, si, di, ra, rb, rc, sem):
        cid = lax.axis_index("c")
        sid = lax.axis_index("s")
        ebase = (cid * NS + sid) * epw

        def chunk(i, carry):
            off = ebase + i * _K
            pltpu.sync_copy(rs_hbm.at[pl.ds(off, _K)], si)
            pltpu.sync_copy(rd_hbm.at[pl.ds(off, _K)], di)
            cpa = pltpu.async_copy(a_hbm.at[si], ra, sem)
            cpb = pltpu.async_copy(b_hbm.at[di], rb, sem)
            pltpu.sync_copy(c_hbm.at[pl.ds(off, _K)], rc)
            cpa.wait()
            cpb.wait()
            _relu_sum3(ra, rb, rc, 8)
            pltpu.sync_copy(ra, g_hbm.at[pl.ds(off, _K)])
            return carry

        lax.fori_loop(0, nch, chunk, 0)

    return k(as_tab, bs_tab, cz_edges, rs, rd)


def kernel(atom_features, atom_edge_index, edge_attr, edge_sh, atom_res_batch,
           res_features, res_edge_index, res_edge_features,
           W1, b1, W2, Wself, Wr1, Wr2, Wr3, Wz1, Wz2, Wm):
    L = W1.shape[0]

    src = atom_edge_index[0].astype(jnp.int32)
    dst = atom_edge_index[1].astype(jnp.int32)
    resb = jnp.pad(atom_res_batch.astype(jnp.int32), (0, NAP - NA))
    rs = jnp.pad(res_edge_index[0].astype(jnp.int32), (0, ERP - ER))
    rd = jnp.pad(res_edge_index[1].astype(jnp.int32), (0, ERP - ER))

    x_pad = jnp.pad(atom_features, ((0, NAP - NA), (0, 0)))
    s = res_features
    z_pad = jnp.pad(res_edge_features, ((0, ERP - ER), (0, 0)))

    z_na = jnp.zeros((NAP, 64), _f32)
    z_r56 = jnp.zeros((NRP, DA), _f32)
    z_r128 = jnp.zeros((NRP, CS), _f32)

    for l in range(L):
        w1 = W1[l]
        ab_tab, xs = _tck_pre(x_pad, w1, Wself[l])
        c_edges = _tck_c(edge_attr, edge_sh, w1[2 * DA:2 * DA + HE],
                         w1[2 * DA + HE:], b1[l][None, :])
        hp = _sc_atom(ab_tab, c_edges, src, dst, z_na)
        x_pad = _tck_x(hp[0], hp[1], xs, W2[l])

        rp = _sc_scatter(x_pad, resb, z_r56, NAP, DA)
        wz1 = Wz1[l]
        s1, as_tab, bs_tab = _tck_res1(s, rp[0], rp[1], Wr1[l], Wr2[l], Wr3[l],
                                       wz1[:CS], wz1[CS:2 * CS])
        cz = _tck_cz(z_pad, wz1[2 * CS:])
        g = _sc_resedge(as_tab, bs_tab, cz, rs, rd)
        z_pad, t = _tck_em(g, z_pad, Wz2[l], Wm[l])
        sp = _sc_scatter(t, rd, z_r128, ERP, CS)
        s = _tck_s(s1, sp[0], sp[1])

    return x_pad[:NA], s, z_pad[:ER]
